# Initial kernel scaffold; baseline (speedup 1.0000x reference)
import jax
import jax.numpy as jnp
from jax.experimental import pallas as pl


def kernel(x, edge_index, edge_weight, W0, W_ih, W_hh, b_ih, b_hh, b_conv, W_fc, b_fc):
    raise NotImplementedError("write your pallas kernel here")


# speedup vs baseline: 20.6000x; 20.6000x over previous
import functools

import jax
import jax.numpy as jnp
from jax import lax
from jax.experimental import pallas as pl
from jax.experimental.pallas import tpu as pltpu
from jax.experimental.pallas import tpu_sc as plsc

N = 10000
E = 320000
D = 128
NC, NS, L = 2, 16, 16
NW = NC * NS
EPT = E // NW
NG = EPT // L
NPAD = 10240
RPC = NPAD // NS
BLK = 1000

_mesh = plsc.VectorSubcoreMesh(core_axis_name="c", subcore_axis_name="s",
                               num_cores=NC, num_subcores=NS)
_scp = pltpu.CompilerParams(needs_layout_passes=False,
                            use_tc_tiling_on_sc=False)


@functools.partial(
    pl.kernel,
    out_type=jax.ShapeDtypeStruct((NW, N), jnp.float32),
    mesh=_mesh,
    compiler_params=_scp,
    scratch_types=[pltpu.VMEM((NG, L), jnp.int32),
                   pltpu.VMEM((NG, L), jnp.float32),
                   pltpu.VMEM((N,), jnp.float32)],
)
def _deg_kernel(dst_hbm, w_hbm, out_hbm, dst_v, w_v, deg_v):
    c = lax.axis_index("c")
    s = lax.axis_index("s")
    wid = c * NS + s
    pltpu.sync_copy(dst_hbm.at[wid], dst_v)
    pltpu.sync_copy(w_hbm.at[wid], w_v)

    @pl.loop(0, N // L)
    def _zero(r):
        deg_v[pl.ds(r * L, L)] = jnp.zeros((L,), jnp.float32)

    @pl.loop(0, NG)
    def _scat(g):
        plsc.addupdate_scatter(deg_v, [dst_v[g]], w_v[g])

    pltpu.sync_copy(deg_v, out_hbm.at[wid])


@functools.partial(
    pl.kernel,
    out_type=jax.ShapeDtypeStruct((NC, NPAD, D), jnp.float32),
    mesh=_mesh,
    compiler_params=_scp,
    scratch_types=[pltpu.VMEM((NG, L), jnp.int32),
                   pltpu.VMEM((NG, L), jnp.int32),
                   pltpu.VMEM((NG, L), jnp.float32),
                   pltpu.VMEM((2, L, D), jnp.float32),
                   pltpu.VMEM((L, D), jnp.float32),
                   pltpu.VMEM_SHARED((NPAD, D), jnp.float32),
                   pltpu.SemaphoreType.DMA((2,))],
)
def _agg_kernel(z_hbm, src_hbm, dst_hbm, w_hbm, out_hbm,
                src_v, dst_v, w_v, rows_v, zer_v, acc_sh, gsem):
    c = lax.axis_index("c")
    s = lax.axis_index("s")
    wid = c * NS + s
    pltpu.sync_copy(src_hbm.at[wid], src_v)
    pltpu.sync_copy(dst_hbm.at[wid], dst_v)
    pltpu.sync_copy(w_hbm.at[wid], w_v)

    for j in range(L):
        for cc in range(D // L):
            zer_v[j, pl.ds(cc * L, L)] = jnp.zeros((L,), jnp.float32)

    base = s * RPC

    @pl.loop(0, RPC // L)
    def _zero(r):
        pltpu.sync_copy(zer_v, acc_sh.at[pl.ds(base + r * L, L)])

    plsc.subcore_barrier()

    def _scale(b, g):
        wv = w_v[g]
        for j in range(L):
            wj = jnp.take_along_axis(wv, jnp.full((L,), j, jnp.int32), axis=0)
            for cc in range(D // L):
                rows_v[b, j, pl.ds(cc * L, L)] = (
                    rows_v[b, j, pl.ds(cc * L, L)] * wj)

    pltpu.async_copy(z_hbm.at[src_v.at[0]], rows_v.at[0], gsem.at[0])

    @pl.loop(0, NG)
    def _edge(g):
        b = lax.rem(g, 2)
        nb = lax.rem(g + 1, 2)

        @pl.when(g + 1 < NG)
        def _():
            pltpu.async_copy(z_hbm.at[src_v.at[g + 1]], rows_v.at[nb],
                             gsem.at[nb])

        pltpu.make_async_copy(z_hbm.at[src_v.at[g]], rows_v.at[b],
                              gsem.at[b]).wait()
        _scale(b, g)
        pltpu.sync_copy(rows_v.at[b], acc_sh.at[dst_v.at[g]], add=True)

    plsc.subcore_barrier()
    pltpu.sync_copy(acc_sh.at[pl.ds(base, RPC)],
                    out_hbm.at[c, pl.ds(base, RPC)])


def _z_body(x_ref, w0_ref, wih_ref, bsum_ref, degt_ref, z_ref, w_s):
    @pl.when(pl.program_id(0) == 0)
    def _():
        gates = lax.dot_general(w0_ref[...], wih_ref[...],
                                (((1,), (1,)), ((), ())),
                                preferred_element_type=jnp.float32)
        gates = gates + bsum_ref[...]
        i_g = gates[:, 0:D]
        g_g = gates[:, 2 * D:3 * D]
        o_g = gates[:, 3 * D:4 * D]
        cst = jax.nn.sigmoid(i_g) * jnp.tanh(g_g)
        w_s[...] = jax.nn.sigmoid(o_g) * jnp.tanh(cst)

    deg = 1.0 + jnp.sum(degt_ref[...], axis=1, keepdims=True)
    dinv = jnp.where(deg > 0, lax.rsqrt(deg), 0.0)
    z_ref[...] = jnp.dot(x_ref[...], w_s[...],
                         preferred_element_type=jnp.float32) * dinv


_z_call = pl.pallas_call(
    _z_body,
    grid=(N // BLK,),
    in_specs=[
        pl.BlockSpec((BLK, D), lambda i: (i, 0)),
        pl.BlockSpec((D, D), lambda i: (0, 0)),
        pl.BlockSpec((4 * D, D), lambda i: (0, 0)),
        pl.BlockSpec((1, 4 * D), lambda i: (0, 0)),
        pl.BlockSpec((BLK, NW), lambda i: (i, 0)),
    ],
    out_specs=pl.BlockSpec((BLK, D), lambda i: (i, 0)),
    out_shape=jax.ShapeDtypeStruct((N, D), jnp.float32),
    scratch_shapes=[pltpu.VMEM((D, D), jnp.float32)],
)


def _head_body(s_ref, z_ref, degt_ref, bconv_ref, wfc_ref, bfc_ref, o_ref):
    deg = 1.0 + jnp.sum(degt_ref[...], axis=1, keepdims=True)
    dinv = jnp.where(deg > 0, lax.rsqrt(deg), 0.0)
    agg = dinv * (s_ref[0] + s_ref[1] + z_ref[...])
    h = jnp.maximum(agg + bconv_ref[...], 0.0)
    o_ref[...] = jnp.dot(h, wfc_ref[...],
                         preferred_element_type=jnp.float32) + bfc_ref[...]


_head_call = pl.pallas_call(
    _head_body,
    grid=(N // BLK,),
    in_specs=[
        pl.BlockSpec((NC, BLK, D), lambda i: (0, i, 0)),
        pl.BlockSpec((BLK, D), lambda i: (i, 0)),
        pl.BlockSpec((BLK, NW), lambda i: (i, 0)),
        pl.BlockSpec((1, D), lambda i: (0, 0)),
        pl.BlockSpec((D, 1), lambda i: (0, 0)),
        pl.BlockSpec((1, 1), lambda i: (0, 0)),
    ],
    out_specs=pl.BlockSpec((BLK, 1), lambda i: (i, 0)),
    out_shape=jax.ShapeDtypeStruct((N, 1), jnp.float32),
)


def kernel(x, edge_index, edge_weight, W0, W_ih, W_hh, b_ih, b_hh, b_conv,
           W_fc, b_fc):
    src = edge_index[0].reshape(NW, NG, L)
    dst = edge_index[1].reshape(NW, NG, L)
    ew = edge_weight.reshape(NW, NG, L)

    deg_part = _deg_kernel(dst, ew)
    deg_t = deg_part.T

    bsum = (b_ih + b_hh).reshape(1, 4 * D)
    z = _z_call(x, W0, W_ih, bsum, deg_t)

    s_part = _agg_kernel(z, src, dst, ew)

    return _head_call(s_part, z, deg_t, b_conv.reshape(1, D),
                      W_fc, b_fc.reshape(1, 1))
